# baseline (speedup 1.0000x reference)
import functools
import math

import jax
import jax.numpy as jnp
from jax import lax
from jax.experimental import pallas as pl
from jax.experimental.pallas import tpu as pltpu
from jax.experimental.pallas import tpu_sc as plsc

B, S, D = 2, 2048, 768
H = 3072
E = 8
K = 2
CAP_F = 1.25
FALLBACK_W = 1.0
N = B * S
C = int(math.ceil(CAP_F * K * N / E))
DUMP = E * C
R = E * C + C

BLK = 512
NB = N // BLK
HH = 768
HT = H // HH
RT = 256
CT = C // RT

NC = 2
NS = 16
NW = NC * NS
CH = 64


def _route_body(x_ref, wr_ref, g_ref,
                w0_ref, w1_ref, dd_ref, dg_ref,
                drop_ref, totc_ref,
                carry_ref, sp0, sp1, se0, se1, sv0, sv1):
    s = pl.program_id(0)

    @pl.when(s == 0)
    def _():
        carry_ref[...] = jnp.zeros_like(carry_ref)

    @pl.when(s < NB)
    def _():
        xs = x_ref[...]
        ms = jnp.mean(xs * xs, axis=1, keepdims=True)
        xn = xs * lax.rsqrt(ms + 1e-6) * g_ref[...]
        logits = jnp.dot(xn.astype(jnp.bfloat16),
                         wr_ref[...].astype(jnp.bfloat16),
                         preferred_element_type=jnp.float32)
        gates = jax.nn.softmax(logits, axis=-1)

        ecols = lax.broadcasted_iota(jnp.int32, (BLK, E), 1)
        v0 = jnp.max(gates, axis=1, keepdims=True)
        e0 = jnp.min(jnp.where(gates == v0, ecols, E), axis=1, keepdims=True)
        gm = jnp.where(ecols == e0, -1.0, gates)
        v1 = jnp.max(gm, axis=1, keepdims=True)
        e1 = jnp.min(jnp.where(gm == v1, ecols, E), axis=1, keepdims=True)

        oh0 = (ecols == e0).astype(jnp.float32)
        oh1 = (ecols == e1).astype(jnp.float32)

        ri = lax.broadcasted_iota(jnp.int32, (BLK, BLK), 0)
        ci = lax.broadcasted_iota(jnp.int32, (BLK, BLK), 1)
        tri = (ci < ri).astype(jnp.bfloat16)
        oh = jnp.concatenate([oh0, oh1], axis=1)
        cum = jnp.dot(tri, oh.astype(jnp.bfloat16),
                      preferred_element_type=jnp.float32)

        c0 = carry_ref[0:1, 0:E]
        c1 = carry_ref[1:2, 0:E]
        sp0[pl.ds(s, 1), :] = jnp.sum((c0 + cum[:, :E]) * oh0, axis=1,
                                      keepdims=True).reshape(1, BLK)
        sp1[pl.ds(s, 1), :] = jnp.sum((c1 + cum[:, E:]) * oh1, axis=1,
                                      keepdims=True).reshape(1, BLK)
        se0[pl.ds(s, 1), :] = e0.reshape(1, BLK)
        se1[pl.ds(s, 1), :] = e1.reshape(1, BLK)
        sv0[pl.ds(s, 1), :] = v0.reshape(1, BLK)
        sv1[pl.ds(s, 1), :] = v1.reshape(1, BLK)
        c0n = c0 + jnp.sum(oh0, axis=0, keepdims=True)
        c1n = c1 + jnp.sum(oh1, axis=0, keepdims=True)
        carry_ref[0:1, 0:E] = c0n
        carry_ref[1:2, 0:E] = c1n
        totc_ref[...] = c0n + c1n

    @pl.when(s >= NB)
    def _():
        b = s - NB
        p0 = sp0[pl.ds(b, 1), :]
        p1 = sp1[pl.ds(b, 1), :]
        e0 = se0[pl.ds(b, 1), :]
        e1 = se1[pl.ds(b, 1), :]
        v0 = sv0[pl.ds(b, 1), :]
        v1 = sv1[pl.ds(b, 1), :]
        tot0 = carry_ref[0:1, 0:E]
        off1 = jnp.zeros_like(p1)
        for e in range(E):
            off1 = jnp.where(e1 == e, tot0[0:1, e:e + 1], off1)
        p1 = p1 + off1
        k0 = p0 < float(C)
        k1 = p1 < float(C)
        w0 = jnp.where(k0, v0, 0.0)
        w1 = jnp.where(k1, v1, 0.0)
        den = jnp.maximum(w0 + w1, 1e-9)
        w0_ref[0, :, :] = w0 / den
        w1_ref[0, :, :] = w1 / den
        s0 = e0 * C + p0.astype(jnp.int32)
        s1 = e1 * C + p1.astype(jnp.int32)
        dd_ref[0, :, 0:BLK] = jnp.where(k0, s0, DUMP)
        dd_ref[0, :, BLK:2 * BLK] = jnp.where(k1, s1, DUMP)
        dg_ref[0, :, 0:BLK] = jnp.where(k0, s0, 0)
        dg_ref[0, :, BLK:2 * BLK] = jnp.where(k1, s1, 0)
        drop_ref[0, :, :] = jnp.logical_and(~k0, ~k1).astype(jnp.float32)


def _route(x_flat, Wr, g2):
    out3 = (NB, 1, BLK)
    out3w = (NB, 1, 2 * BLK)
    blk = pl.BlockSpec((1, 1, BLK), lambda s: (s % NB, 0, 0))
    blkw = pl.BlockSpec((1, 1, 2 * BLK), lambda s: (s % NB, 0, 0))
    return pl.pallas_call(
        _route_body,
        grid=(2 * NB,),
        in_specs=[
            pl.BlockSpec((BLK, D), lambda s: (jnp.minimum(s, NB - 1), 0)),
            pl.BlockSpec((D, E), lambda s: (0, 0)),
            pl.BlockSpec((1, D), lambda s: (0, 0)),
        ],
        out_specs=[blk, blk, blkw, blkw, blk,
                   pl.BlockSpec((1, E), lambda s: (0, 0))],
        out_shape=[
            jax.ShapeDtypeStruct(out3, jnp.float32),
            jax.ShapeDtypeStruct(out3, jnp.float32),
            jax.ShapeDtypeStruct(out3w, jnp.int32),
            jax.ShapeDtypeStruct(out3w, jnp.int32),
            jax.ShapeDtypeStruct(out3, jnp.float32),
            jax.ShapeDtypeStruct((1, E), jnp.float32),
        ],
        scratch_shapes=[
            pltpu.VMEM((8, 128), jnp.float32),
            pltpu.VMEM((NB, BLK), jnp.float32),
            pltpu.VMEM((NB, BLK), jnp.float32),
            pltpu.VMEM((NB, BLK), jnp.int32),
            pltpu.VMEM((NB, BLK), jnp.int32),
            pltpu.VMEM((NB, BLK), jnp.float32),
            pltpu.VMEM((NB, BLK), jnp.float32),
        ],
        compiler_params=pltpu.CompilerParams(
            dimension_semantics=("arbitrary",)),
    )(x_flat, Wr, g2)


def _sc_dispatch(x_flat, dd):
    mesh = plsc.VectorSubcoreMesh(core_axis_name="c", subcore_axis_name="s")
    n_ch = K * (N // NW) // CH

    @functools.partial(
        pl.kernel,
        out_type=jax.ShapeDtypeStruct((R, D), jnp.float32),
        mesh=mesh,
        scratch_types=[
            [pltpu.VMEM((CH,), jnp.int32) for _ in range(n_ch)],
            [pltpu.VMEM((CH, D), jnp.float32) for _ in range(2)],
            [pltpu.SemaphoreType.DMA for _ in range(2)],
        ],
    )
    def disp(x_hbm, dd_hbm, xec_hbm, idx_v, rows_v, sems):
        wid = lax.axis_index("s") * NC + lax.axis_index("c")
        per_w = N // NW
        chunks = [(k, c) for k in range(K) for c in range(per_w // CH)]
        for j, (k, c) in enumerate(chunks):
            base = wid * per_w + c * CH
            blk = base // BLK
            off = base - blk * BLK
            pltpu.sync_copy(dd_hbm.at[blk, 0, pl.ds(k * BLK + off, CH)],
                            idx_v[j])
        pending = [None, None]
        for j, (k, c) in enumerate(chunks):
            buf = j % 2
            if pending[buf] is not None:
                pending[buf].wait()
            base = wid * per_w + c * CH
            pltpu.sync_copy(x_hbm.at[pl.ds(base, CH)], rows_v[buf])
            cp = pltpu.make_async_copy(rows_v[buf], xec_hbm.at[idx_v[j]],
                                       sems[buf])
            cp.start()
            pending[buf] = cp
        for cp in pending:
            cp.wait()

    return disp(x_flat, dd)


def _ffn_body(xec_ref, w1_ref, b1_ref, w2_ref, b2_ref, totc_ref, y_ref,
              stg1_ref, stg2_ref, w1s_ref, w2s_ref, sm1, sm2):
    e = pl.program_id(0)
    c = pl.program_id(1)
    cnt = totc_ref[0, e]

    @pl.when(jnp.logical_and(e == 0, c == 0))
    def _():
        pltpu.make_async_copy(w1_ref.at[0], stg1_ref, sm1).start()
        pltpu.make_async_copy(w2_ref.at[0], stg2_ref, sm2).start()

    @pl.when(c == 0)
    def _():
        pltpu.make_async_copy(w1_ref.at[e], stg1_ref, sm1).wait()
        pltpu.make_async_copy(w2_ref.at[e], stg2_ref, sm2).wait()

        @pl.when(cnt > 0.0)
        def _():
            w1s_ref[...] = stg1_ref[...].astype(jnp.bfloat16)
            w2s_ref[...] = stg2_ref[...].astype(jnp.bfloat16)

    @pl.when(jnp.logical_and(c == 1, e < E - 1))
    def _():
        pltpu.make_async_copy(w1_ref.at[e + 1], stg1_ref, sm1).start()
        pltpu.make_async_copy(w2_ref.at[e + 1], stg2_ref, sm2).start()

    @pl.when(c * RT < cnt)
    def _():
        xb = xec_ref[...].astype(jnp.bfloat16)
        acc = jnp.zeros((RT, D), jnp.float32) + b2_ref[0, 0, :]
        for t in range(HT):
            lo = t * HH
            a = jnp.dot(xb, w1s_ref[:, lo:lo + HH],
                        preferred_element_type=jnp.float32)
            a = a + b1_ref[0, 0, lo:lo + HH]
            bb = jnp.dot(xb, w1s_ref[:, H + lo:H + lo + HH],
                         preferred_element_type=jnp.float32)
            bb = bb + b1_ref[0, 0, H + lo:H + lo + HH]
            g = (a * jax.nn.sigmoid(a)) * bb
            acc = acc + jnp.dot(g.astype(jnp.bfloat16),
                                w2s_ref[lo:lo + HH, :],
                                preferred_element_type=jnp.float32)
        y_ref[...] = acc


def _ffn(x_ec, W1, b1, W2, b2, totc):
    return pl.pallas_call(
        _ffn_body,
        grid=(E, CT),
        in_specs=[
            pl.BlockSpec((RT, D), lambda e, c: (e * CT + c, 0)),
            pl.BlockSpec(memory_space=pl.ANY),
            pl.BlockSpec((1, 1, 2 * H), lambda e, c: (e, 0, 0)),
            pl.BlockSpec(memory_space=pl.ANY),
            pl.BlockSpec((1, 1, D), lambda e, c: (e, 0, 0)),
            pl.BlockSpec(memory_space=pltpu.SMEM),
        ],
        out_specs=pl.BlockSpec((RT, D), lambda e, c: (e * CT + c, 0)),
        out_shape=jax.ShapeDtypeStruct((E * C, D), jnp.float32),
        scratch_shapes=[
            pltpu.VMEM((D, 2 * H), jnp.float32),
            pltpu.VMEM((H, D), jnp.float32),
            pltpu.VMEM((D, 2 * H), jnp.bfloat16),
            pltpu.VMEM((H, D), jnp.bfloat16),
            pltpu.SemaphoreType.DMA,
            pltpu.SemaphoreType.DMA,
        ],
        compiler_params=pltpu.CompilerParams(
            dimension_semantics=("arbitrary", "arbitrary")),
    )(x_ec, W1, b1.reshape(E, 1, 2 * H), W2, b2.reshape(E, 1, D), totc)


def _sc_gather(y_ec, dg):
    mesh = plsc.VectorSubcoreMesh(core_axis_name="c", subcore_axis_name="s")
    n_ch = K * (N // NW) // CH

    @functools.partial(
        pl.kernel,
        out_type=jax.ShapeDtypeStruct((K, N, D), jnp.float32),
        mesh=mesh,
        scratch_types=[
            [pltpu.VMEM((CH,), jnp.int32) for _ in range(n_ch)],
            [pltpu.VMEM((CH, D), jnp.float32) for _ in range(2)],
            [pltpu.SemaphoreType.DMA for _ in range(2)],
        ],
    )
    def gath(yec_hbm, dg_hbm, yg_hbm, idx_v, rows_v, sems):
        wid = lax.axis_index("s") * NC + lax.axis_index("c")
        per_w = N // NW
        chunks = [(k, c) for k in range(K) for c in range(per_w // CH)]
        for j, (k, c) in enumerate(chunks):
            base = wid * per_w + c * CH
            blk = base // BLK
            off = base - blk * BLK
            pltpu.sync_copy(dg_hbm.at[blk, 0, pl.ds(k * BLK + off, CH)],
                            idx_v[j])
        pending = [None, None]

        def start(j):
            cp = pltpu.make_async_copy(yec_hbm.at[idx_v[j]], rows_v[j % 2],
                                       sems[j % 2])
            cp.start()
            pending[j % 2] = cp

        start(0)
        for j, (k, c) in enumerate(chunks):
            pending[j % 2].wait()
            if j + 1 < len(chunks):
                start(j + 1)
            base = wid * per_w + c * CH
            pltpu.sync_copy(rows_v[j % 2], yg_hbm.at[k, pl.ds(base, CH)])

    return gath(y_ec, dg)


def _final_body(y0_ref, y1_ref, w0_ref, w1_ref, drop_ref, x_ref,
                fw1_ref, fw2_ref, fb1_ref, fb2_ref, out_ref,
                xs_ref, wa_ref, wb_ref, w2s_ref, s0, s1, s2, s3):
    b = pl.program_id(0)
    w0 = w0_ref[...]
    w1 = w1_ref[...]
    y0 = y0_ref[...]
    y1 = y1_ref[...]
    out_ref[...] = (jnp.where(w0 != 0.0, w0 * y0, 0.0)
                    + jnp.where(w1 != 0.0, w1 * y1, 0.0))

    dropv = drop_ref[...]

    @pl.when(jnp.sum(dropv) > 0.0)
    def _():
        cx = pltpu.make_async_copy(
            x_ref.at[pl.ds(b * BLK, BLK), :], xs_ref, s3)
        cx.start()
        cx.wait()
        xb = xs_ref[...].astype(jnp.bfloat16)
        acc = jnp.zeros((BLK, D), jnp.float32)
        for t in range(HT):
            c1 = pltpu.make_async_copy(
                fw1_ref.at[:, pl.ds(t * HH, HH)], wa_ref, s0)
            c2 = pltpu.make_async_copy(
                fw1_ref.at[:, pl.ds(H + t * HH, HH)], wb_ref, s1)
            c3 = pltpu.make_async_copy(
                fw2_ref.at[pl.ds(t * HH, HH), :], w2s_ref, s2)
            c1.start()
            c2.start()
            c3.start()
            c1.wait()
            c2.wait()
            c3.wait()
            a = jnp.dot(xb, wa_ref[...].astype(jnp.bfloat16),
                        preferred_element_type=jnp.float32)
            a = a + fb1_ref[0:1, t * HH:(t + 1) * HH]
            bb = jnp.dot(xb, wb_ref[...].astype(jnp.bfloat16),
                         preferred_element_type=jnp.float32)
            bb = bb + fb1_ref[0:1, H + t * HH:H + (t + 1) * HH]
            g = (a * jax.nn.sigmoid(a)) * bb
            acc = acc + jnp.dot(g.astype(jnp.bfloat16),
                                w2s_ref[...].astype(jnp.bfloat16),
                                preferred_element_type=jnp.float32)
        acc = acc + fb2_ref[...]
        out_ref[...] += jnp.where(dropv != 0.0, FALLBACK_W * acc, 0.0)


def _final(y_g2, w0c, w1c, dropc, x_flat, fW1b, fW2b, fb1_2, fb2_2):
    return pl.pallas_call(
        _final_body,
        grid=(NB,),
        in_specs=[
            pl.BlockSpec((BLK, D), lambda b: (b, 0)),
            pl.BlockSpec((BLK, D), lambda b: (b + NB, 0)),
            pl.BlockSpec((BLK, 1), lambda b: (b, 0)),
            pl.BlockSpec((BLK, 1), lambda b: (b, 0)),
            pl.BlockSpec((BLK, 1), lambda b: (b, 0)),
            pl.BlockSpec(memory_space=pl.ANY),
            pl.BlockSpec(memory_space=pl.ANY),
            pl.BlockSpec(memory_space=pl.ANY),
            pl.BlockSpec((1, 2 * H), lambda b: (0, 0)),
            pl.BlockSpec((1, D), lambda b: (0, 0)),
        ],
        out_specs=pl.BlockSpec((BLK, D), lambda b: (b, 0)),
        out_shape=jax.ShapeDtypeStruct((N, D), jnp.float32),
        scratch_shapes=[
            pltpu.VMEM((BLK, D), jnp.float32),
            pltpu.VMEM((D, HH), jnp.float32),
            pltpu.VMEM((D, HH), jnp.float32),
            pltpu.VMEM((HH, D), jnp.float32),
            pltpu.SemaphoreType.DMA,
            pltpu.SemaphoreType.DMA,
            pltpu.SemaphoreType.DMA,
            pltpu.SemaphoreType.DMA,
        ],
    )(y_g2, y_g2, w0c, w1c, dropc, x_flat, fW1b, fW2b, fb1_2, fb2_2)


def kernel(x, Wr, rms_g, W1, b1, W2, b2, fW1, fb1, fW2, fb2):
    x_flat = x.reshape(N, D)
    g2 = rms_g.reshape(1, D)

    w0b, w1b, dd, dg, dropb, totc = _route(x_flat, Wr, g2)

    x_ec = _sc_dispatch(x_flat, dd)

    y_ec = _ffn(x_ec, W1, b1, W2, b2, totc)

    y_g = _sc_gather(y_ec, dg)

    out = _final(
        y_g.reshape(K * N, D),
        w0b.reshape(N, 1),
        w1b.reshape(N, 1),
        dropb.reshape(N, 1),
        x_flat,
        fW1,
        fW2,
        fb1.reshape(1, 2 * H),
        fb2.reshape(1, D),
    )
    return out.reshape(B, S, D)


# speedup vs baseline: 1.0151x; 1.0151x over previous
import functools
import math

import jax
import jax.numpy as jnp
from jax import lax
from jax.experimental import pallas as pl
from jax.experimental.pallas import tpu as pltpu
from jax.experimental.pallas import tpu_sc as plsc

B, S, D = 2, 2048, 768
H = 3072
E = 8
K = 2
CAP_F = 1.25
FALLBACK_W = 1.0
N = B * S
C = int(math.ceil(CAP_F * K * N / E))
DUMP = E * C
R = E * C + C

BLK = 512
NB = N // BLK
HH = 768
HT = H // HH
RT = 256
CT = C // RT

NC = 2
NS = 16
NW = NC * NS
CH = 64


def _route_body(x_ref, wr_ref, g_ref,
                w0_ref, w1_ref, dd_ref, dg_ref,
                drop_ref, totc_ref,
                carry_ref, sp0, sp1, se0, se1, sv0, sv1):
    s = pl.program_id(0)

    @pl.when(s == 0)
    def _():
        carry_ref[...] = jnp.zeros_like(carry_ref)

    @pl.when(s < NB)
    def _():
        xs = x_ref[...]
        ms = jnp.mean(xs * xs, axis=1, keepdims=True)
        xn = xs * lax.rsqrt(ms + 1e-6) * g_ref[...]
        logits = jnp.dot(xn, wr_ref[...],
                         preferred_element_type=jnp.float32)
        gates = jax.nn.softmax(logits, axis=-1)

        ecols = lax.broadcasted_iota(jnp.int32, (BLK, E), 1)
        v0 = jnp.max(gates, axis=1, keepdims=True)
        e0 = jnp.min(jnp.where(gates == v0, ecols, E), axis=1, keepdims=True)
        gm = jnp.where(ecols == e0, -1.0, gates)
        v1 = jnp.max(gm, axis=1, keepdims=True)
        e1 = jnp.min(jnp.where(gm == v1, ecols, E), axis=1, keepdims=True)

        oh0 = (ecols == e0).astype(jnp.float32)
        oh1 = (ecols == e1).astype(jnp.float32)

        ri = lax.broadcasted_iota(jnp.int32, (BLK, BLK), 0)
        ci = lax.broadcasted_iota(jnp.int32, (BLK, BLK), 1)
        tri = (ci < ri).astype(jnp.float32)
        oh = jnp.concatenate([oh0, oh1], axis=1)
        cum = jnp.dot(tri, oh, preferred_element_type=jnp.float32)

        c0 = carry_ref[0:1, 0:E]
        c1 = carry_ref[1:2, 0:E]
        sp0[pl.ds(s, 1), :] = jnp.sum((c0 + cum[:, :E]) * oh0, axis=1,
                                      keepdims=True).reshape(1, BLK)
        sp1[pl.ds(s, 1), :] = jnp.sum((c1 + cum[:, E:]) * oh1, axis=1,
                                      keepdims=True).reshape(1, BLK)
        se0[pl.ds(s, 1), :] = e0.reshape(1, BLK)
        se1[pl.ds(s, 1), :] = e1.reshape(1, BLK)
        sv0[pl.ds(s, 1), :] = v0.reshape(1, BLK)
        sv1[pl.ds(s, 1), :] = v1.reshape(1, BLK)
        c0n = c0 + jnp.sum(oh0, axis=0, keepdims=True)
        c1n = c1 + jnp.sum(oh1, axis=0, keepdims=True)
        carry_ref[0:1, 0:E] = c0n
        carry_ref[1:2, 0:E] = c1n
        totc_ref[...] = c0n + c1n

    @pl.when(s >= NB)
    def _():
        b = s - NB
        p0 = sp0[pl.ds(b, 1), :]
        p1 = sp1[pl.ds(b, 1), :]
        e0 = se0[pl.ds(b, 1), :]
        e1 = se1[pl.ds(b, 1), :]
        v0 = sv0[pl.ds(b, 1), :]
        v1 = sv1[pl.ds(b, 1), :]
        tot0 = carry_ref[0:1, 0:E]
        off1 = jnp.zeros_like(p1)
        for e in range(E):
            off1 = jnp.where(e1 == e, tot0[0:1, e:e + 1], off1)
        p1 = p1 + off1
        k0 = p0 < float(C)
        k1 = p1 < float(C)
        w0 = jnp.where(k0, v0, 0.0)
        w1 = jnp.where(k1, v1, 0.0)
        den = jnp.maximum(w0 + w1, 1e-9)
        w0_ref[0, :, :] = w0 / den
        w1_ref[0, :, :] = w1 / den
        s0 = e0 * C + p0.astype(jnp.int32)
        s1 = e1 * C + p1.astype(jnp.int32)
        dd_ref[0, :, 0:BLK] = jnp.where(k0, s0, DUMP)
        dd_ref[0, :, BLK:2 * BLK] = jnp.where(k1, s1, DUMP)
        dg_ref[0, :, 0:BLK] = jnp.where(k0, s0, 0)
        dg_ref[0, :, BLK:2 * BLK] = jnp.where(k1, s1, 0)
        drop_ref[0, :, :] = jnp.logical_and(~k0, ~k1).astype(jnp.float32)


def _route(x_flat, Wr, g2):
    out3 = (NB, 1, BLK)
    out3w = (NB, 1, 2 * BLK)
    blk = pl.BlockSpec((1, 1, BLK), lambda s: (s % NB, 0, 0))
    blkw = pl.BlockSpec((1, 1, 2 * BLK), lambda s: (s % NB, 0, 0))
    return pl.pallas_call(
        _route_body,
        grid=(2 * NB,),
        in_specs=[
            pl.BlockSpec((BLK, D), lambda s: (jnp.minimum(s, NB - 1), 0)),
            pl.BlockSpec((D, E), lambda s: (0, 0)),
            pl.BlockSpec((1, D), lambda s: (0, 0)),
        ],
        out_specs=[blk, blk, blkw, blkw, blk,
                   pl.BlockSpec((1, E), lambda s: (0, 0))],
        out_shape=[
            jax.ShapeDtypeStruct(out3, jnp.float32),
            jax.ShapeDtypeStruct(out3, jnp.float32),
            jax.ShapeDtypeStruct(out3w, jnp.int32),
            jax.ShapeDtypeStruct(out3w, jnp.int32),
            jax.ShapeDtypeStruct(out3, jnp.float32),
            jax.ShapeDtypeStruct((1, E), jnp.float32),
        ],
        scratch_shapes=[
            pltpu.VMEM((8, 128), jnp.float32),
            pltpu.VMEM((NB, BLK), jnp.float32),
            pltpu.VMEM((NB, BLK), jnp.float32),
            pltpu.VMEM((NB, BLK), jnp.int32),
            pltpu.VMEM((NB, BLK), jnp.int32),
            pltpu.VMEM((NB, BLK), jnp.float32),
            pltpu.VMEM((NB, BLK), jnp.float32),
        ],
        compiler_params=pltpu.CompilerParams(
            dimension_semantics=("arbitrary",)),
    )(x_flat, Wr, g2)


def _sc_dispatch(x_flat, dd):
    mesh = plsc.VectorSubcoreMesh(core_axis_name="c", subcore_axis_name="s")
    n_ch = K * (N // NW) // CH

    @functools.partial(
        pl.kernel,
        out_type=jax.ShapeDtypeStruct((R, D), jnp.float32),
        mesh=mesh,
        scratch_types=[
            [pltpu.VMEM((CH,), jnp.int32) for _ in range(n_ch)],
            [pltpu.VMEM((CH, D), jnp.float32) for _ in range(2)],
            [pltpu.SemaphoreType.DMA for _ in range(2)],
        ],
    )
    def disp(x_hbm, dd_hbm, xec_hbm, idx_v, rows_v, sems):
        wid = lax.axis_index("s") * NC + lax.axis_index("c")
        per_w = N // NW
        chunks = [(k, c) for k in range(K) for c in range(per_w // CH)]
        for j, (k, c) in enumerate(chunks):
            base = wid * per_w + c * CH
            blk = base // BLK
            off = base - blk * BLK
            pltpu.sync_copy(dd_hbm.at[blk, 0, pl.ds(k * BLK + off, CH)],
                            idx_v[j])
        pending = [None, None]
        for j, (k, c) in enumerate(chunks):
            buf = j % 2
            if pending[buf] is not None:
                pending[buf].wait()
            base = wid * per_w + c * CH
            pltpu.sync_copy(x_hbm.at[pl.ds(base, CH)], rows_v[buf])
            cp = pltpu.make_async_copy(rows_v[buf], xec_hbm.at[idx_v[j]],
                                       sems[buf])
            cp.start()
            pending[buf] = cp
        for cp in pending:
            cp.wait()

    return disp(x_flat, dd)


def _ffn_body(xec_ref, w1_ref, b1_ref, w2_ref, b2_ref, totc_ref, y_ref,
              stg1_ref, stg2_ref, w1s_ref, w2s_ref, sm1, sm2):
    e = pl.program_id(0)
    c = pl.program_id(1)
    cnt = totc_ref[0, e]

    @pl.when(jnp.logical_and(e == 0, c == 0))
    def _():
        pltpu.make_async_copy(w1_ref.at[0], stg1_ref, sm1).start()
        pltpu.make_async_copy(w2_ref.at[0], stg2_ref, sm2).start()

    @pl.when(c == 0)
    def _():
        pltpu.make_async_copy(w1_ref.at[e], stg1_ref, sm1).wait()
        pltpu.make_async_copy(w2_ref.at[e], stg2_ref, sm2).wait()

        @pl.when(cnt > 0.0)
        def _():
            w1s_ref[...] = stg1_ref[...].astype(jnp.bfloat16)
            w2s_ref[...] = stg2_ref[...].astype(jnp.bfloat16)

    @pl.when(jnp.logical_and(c == 1, e < E - 1))
    def _():
        pltpu.make_async_copy(w1_ref.at[e + 1], stg1_ref, sm1).start()
        pltpu.make_async_copy(w2_ref.at[e + 1], stg2_ref, sm2).start()

    @pl.when(c * RT < cnt)
    def _():
        xb = xec_ref[...].astype(jnp.bfloat16)
        acc = jnp.zeros((RT, D), jnp.float32) + b2_ref[0, 0, :]
        for t in range(HT):
            lo = t * HH
            a = jnp.dot(xb, w1s_ref[:, lo:lo + HH],
                        preferred_element_type=jnp.float32)
            a = a + b1_ref[0, 0, lo:lo + HH]
            bb = jnp.dot(xb, w1s_ref[:, H + lo:H + lo + HH],
                         preferred_element_type=jnp.float32)
            bb = bb + b1_ref[0, 0, H + lo:H + lo + HH]
            g = (a * jax.nn.sigmoid(a)) * bb
            acc = acc + jnp.dot(g.astype(jnp.bfloat16),
                                w2s_ref[lo:lo + HH, :],
                                preferred_element_type=jnp.float32)
        y_ref[...] = acc


def _ffn(x_ec, W1, b1, W2, b2, totc):
    return pl.pallas_call(
        _ffn_body,
        grid=(E, CT),
        in_specs=[
            pl.BlockSpec((RT, D), lambda e, c: (e * CT + c, 0)),
            pl.BlockSpec(memory_space=pl.ANY),
            pl.BlockSpec((1, 1, 2 * H), lambda e, c: (e, 0, 0)),
            pl.BlockSpec(memory_space=pl.ANY),
            pl.BlockSpec((1, 1, D), lambda e, c: (e, 0, 0)),
            pl.BlockSpec(memory_space=pltpu.SMEM),
        ],
        out_specs=pl.BlockSpec((RT, D), lambda e, c: (e * CT + c, 0)),
        out_shape=jax.ShapeDtypeStruct((E * C, D), jnp.float32),
        scratch_shapes=[
            pltpu.VMEM((D, 2 * H), jnp.float32),
            pltpu.VMEM((H, D), jnp.float32),
            pltpu.VMEM((D, 2 * H), jnp.bfloat16),
            pltpu.VMEM((H, D), jnp.bfloat16),
            pltpu.SemaphoreType.DMA,
            pltpu.SemaphoreType.DMA,
        ],
        compiler_params=pltpu.CompilerParams(
            dimension_semantics=("arbitrary", "arbitrary")),
    )(x_ec, W1, b1.reshape(E, 1, 2 * H), W2, b2.reshape(E, 1, D), totc)


def _sc_gather(y_ec, dg):
    mesh = plsc.VectorSubcoreMesh(core_axis_name="c", subcore_axis_name="s")
    n_ch = K * (N // NW) // CH

    @functools.partial(
        pl.kernel,
        out_type=jax.ShapeDtypeStruct((K, N, D), jnp.float32),
        mesh=mesh,
        scratch_types=[
            [pltpu.VMEM((CH,), jnp.int32) for _ in range(n_ch)],
            [pltpu.VMEM((CH, D), jnp.float32) for _ in range(2)],
            [pltpu.SemaphoreType.DMA for _ in range(2)],
        ],
    )
    def gath(yec_hbm, dg_hbm, yg_hbm, idx_v, rows_v, sems):
        wid = lax.axis_index("s") * NC + lax.axis_index("c")
        per_w = N // NW
        chunks = [(k, c) for k in range(K) for c in range(per_w // CH)]
        for j, (k, c) in enumerate(chunks):
            base = wid * per_w + c * CH
            blk = base // BLK
            off = base - blk * BLK
            pltpu.sync_copy(dg_hbm.at[blk, 0, pl.ds(k * BLK + off, CH)],
                            idx_v[j])
        pending = [None, None]

        def start(j):
            cp = pltpu.make_async_copy(yec_hbm.at[idx_v[j]], rows_v[j % 2],
                                       sems[j % 2])
            cp.start()
            pending[j % 2] = cp

        start(0)
        for j, (k, c) in enumerate(chunks):
            pending[j % 2].wait()
            if j + 1 < len(chunks):
                start(j + 1)
            base = wid * per_w + c * CH
            pltpu.sync_copy(rows_v[j % 2], yg_hbm.at[k, pl.ds(base, CH)])

    return gath(y_ec, dg)


def _final_body(y0_ref, y1_ref, w0_ref, w1_ref, drop_ref, x_ref,
                fw1_ref, fw2_ref, fb1_ref, fb2_ref, out_ref,
                xs_ref, wa_ref, wb_ref, w2s_ref, s0, s1, s2, s3):
    b = pl.program_id(0)
    w0 = w0_ref[...]
    w1 = w1_ref[...]
    y0 = y0_ref[...]
    y1 = y1_ref[...]
    out_ref[...] = (jnp.where(w0 != 0.0, w0 * y0, 0.0)
                    + jnp.where(w1 != 0.0, w1 * y1, 0.0))

    dropv = drop_ref[...]

    @pl.when(jnp.sum(dropv) > 0.0)
    def _():
        cx = pltpu.make_async_copy(
            x_ref.at[pl.ds(b * BLK, BLK), :], xs_ref, s3)
        cx.start()
        cx.wait()
        xb = xs_ref[...].astype(jnp.bfloat16)
        acc = jnp.zeros((BLK, D), jnp.float32)
        for t in range(HT):
            c1 = pltpu.make_async_copy(
                fw1_ref.at[:, pl.ds(t * HH, HH)], wa_ref, s0)
            c2 = pltpu.make_async_copy(
                fw1_ref.at[:, pl.ds(H + t * HH, HH)], wb_ref, s1)
            c3 = pltpu.make_async_copy(
                fw2_ref.at[pl.ds(t * HH, HH), :], w2s_ref, s2)
            c1.start()
            c2.start()
            c3.start()
            c1.wait()
            c2.wait()
            c3.wait()
            a = jnp.dot(xb, wa_ref[...].astype(jnp.bfloat16),
                        preferred_element_type=jnp.float32)
            a = a + fb1_ref[0:1, t * HH:(t + 1) * HH]
            bb = jnp.dot(xb, wb_ref[...].astype(jnp.bfloat16),
                         preferred_element_type=jnp.float32)
            bb = bb + fb1_ref[0:1, H + t * HH:H + (t + 1) * HH]
            g = (a * jax.nn.sigmoid(a)) * bb
            acc = acc + jnp.dot(g.astype(jnp.bfloat16),
                                w2s_ref[...].astype(jnp.bfloat16),
                                preferred_element_type=jnp.float32)
        acc = acc + fb2_ref[...]
        out_ref[...] += jnp.where(dropv != 0.0, FALLBACK_W * acc, 0.0)


def _final(y_g2, w0c, w1c, dropc, x_flat, fW1b, fW2b, fb1_2, fb2_2):
    return pl.pallas_call(
        _final_body,
        grid=(NB,),
        in_specs=[
            pl.BlockSpec((BLK, D), lambda b: (b, 0)),
            pl.BlockSpec((BLK, D), lambda b: (b + NB, 0)),
            pl.BlockSpec((BLK, 1), lambda b: (b, 0)),
            pl.BlockSpec((BLK, 1), lambda b: (b, 0)),
            pl.BlockSpec((BLK, 1), lambda b: (b, 0)),
            pl.BlockSpec(memory_space=pl.ANY),
            pl.BlockSpec(memory_space=pl.ANY),
            pl.BlockSpec(memory_space=pl.ANY),
            pl.BlockSpec((1, 2 * H), lambda b: (0, 0)),
            pl.BlockSpec((1, D), lambda b: (0, 0)),
        ],
        out_specs=pl.BlockSpec((BLK, D), lambda b: (b, 0)),
        out_shape=jax.ShapeDtypeStruct((N, D), jnp.float32),
        scratch_shapes=[
            pltpu.VMEM((BLK, D), jnp.float32),
            pltpu.VMEM((D, HH), jnp.float32),
            pltpu.VMEM((D, HH), jnp.float32),
            pltpu.VMEM((HH, D), jnp.float32),
            pltpu.SemaphoreType.DMA,
            pltpu.SemaphoreType.DMA,
            pltpu.SemaphoreType.DMA,
            pltpu.SemaphoreType.DMA,
        ],
    )(y_g2, y_g2, w0c, w1c, dropc, x_flat, fW1b, fW2b, fb1_2, fb2_2)


def kernel(x, Wr, rms_g, W1, b1, W2, b2, fW1, fb1, fW2, fb2):
    x_flat = x.reshape(N, D)
    g2 = rms_g.reshape(1, D)

    w0b, w1b, dd, dg, dropb, totc = _route(x_flat, Wr, g2)

    x_ec = _sc_dispatch(x_flat, dd)

    y_ec = _ffn(x_ec, W1, b1, W2, b2, totc)

    y_g = _sc_gather(y_ec, dg)

    out = _final(
        y_g.reshape(K * N, D),
        w0b.reshape(N, 1),
        w1b.reshape(N, 1),
        dropb.reshape(N, 1),
        x_flat,
        fW1,
        fW2,
        fb1.reshape(1, 2 * H),
        fb2.reshape(1, D),
    )
    return out.reshape(B, S, D)
